# baseline (speedup 1.0000x reference)
import functools

import jax
import jax.numpy as jnp
from jax import lax
from jax.experimental import pallas as pl
from jax.experimental.pallas import tpu as pltpu
from jax.experimental.pallas import tpu_sc as plsc

BATCH = 16384
HIST = 50
EMBED_DIM = 64
NUM_CORES = 2
NUM_SUBCORES = 16
NUM_WORKERS = NUM_CORES * NUM_SUBCORES
BWIN = BATCH // NUM_WORKERS
CHUNK = 128
SUBT = BWIN // CHUNK
CPW = HIST * SUBT
BTILES = BATCH // CHUNK
SCALE = 8.0
LANES = 16
NBUF = 8
AHEAD = 6
TBUF = 4


@functools.cache
def _build():
    mesh = plsc.VectorSubcoreMesh(core_axis_name="c", subcore_axis_name="s")

    @functools.partial(
        pl.kernel,
        mesh=mesh,
        out_type=jax.ShapeDtypeStruct(
            (HIST, EMBED_DIM // 8, BTILES, 8, CHUNK), jnp.float32
        ),
        scratch_types=[
            pltpu.VMEM((CPW, CHUNK), jnp.int32),
            pltpu.VMEM((NBUF, CHUNK, EMBED_DIM), jnp.float32),
            pltpu.VMEM((TBUF, 8, 8, CHUNK), jnp.float32),
            pltpu.SemaphoreType.DMA,
            pltpu.SemaphoreType.DMA((NBUF,)),
            pltpu.SemaphoreType.DMA((TBUF,)),
        ],
        compiler_params=pltpu.CompilerParams(
            use_tc_tiling_on_sc=False, needs_layout_passes=False
        ),
    )
    def _gather_scale(xt_hbm, tab_hbm, out_hbm, idx_v, rows_v, tbuf_v,
                      isem, gsem, ssem):
        wid = lax.axis_index("s") * NUM_CORES + lax.axis_index("c")
        b_lo = wid * BWIN

        def stage(g, _):
            h = g // SUBT
            b0 = b_lo + lax.rem(g, SUBT) * CHUNK
            pltpu.async_copy(xt_hbm.at[h, pl.ds(b0, CHUNK)], idx_v.at[g], isem)
            return _

        lax.fori_loop(0, CPW, stage, 0)

        def stage_wait(g, _):
            pltpu.make_async_copy(
                xt_hbm.at[0, pl.ds(0, CHUNK)], idx_v.at[0], isem
            ).wait()
            return _

        lax.fori_loop(0, CPW, stage_wait, 0)

        def gather(g, b):
            pltpu.async_copy(tab_hbm.at[idx_v.at[g]], rows_v.at[b], gsem.at[b])

        for b in range(AHEAD):
            gather(b, b)

        lane_ids = [lax.iota(jnp.int32, LANES) + kb * LANES
                    for kb in range(CHUNK // LANES)]

        def outer(go):
            for b in range(NBUF):
                g = go + b
                h = g // SUBT
                btile = wid * SUBT + lax.rem(g, SUBT)
                tb = b % TBUF

                pltpu.make_async_copy(
                    tab_hbm.at[idx_v.at[g]], rows_v.at[b], gsem.at[b]
                ).wait()

                @pl.when(g >= TBUF)
                def _():
                    gp = g - TBUF
                    pltpu.make_async_copy(
                        tbuf_v.at[tb],
                        out_hbm.at[gp // SUBT, :,
                                   wid * SUBT + lax.rem(gp, SUBT)],
                        ssem.at[tb],
                    ).wait()

                def col_body(d, c2, _b=b, _tb=tb):
                    cols = jnp.full((LANES,), d, jnp.int32)
                    dt = d // 8
                    dl = lax.rem(d, 8)
                    for kb in range(CHUNK // LANES):
                        v = plsc.load_gather(
                            rows_v.at[_b], [lane_ids[kb], cols]
                        )
                        tbuf_v[_tb, dt, dl, pl.ds(kb * LANES, LANES)] = (
                            v * SCALE
                        )
                    return c2

                lax.fori_loop(0, EMBED_DIM, col_body, 0, unroll=2)

                pltpu.async_copy(
                    tbuf_v.at[tb], out_hbm.at[h, :, btile], ssem.at[tb]
                )

                gn = g + AHEAD

                @pl.when(gn < CPW)
                def _():
                    gather(gn, (b + AHEAD) % NBUF)

        pl.loop(0, CPW, step=NBUF)(outer)

        for t in range(TBUF):
            g = CPW - TBUF + t
            pltpu.make_async_copy(
                tbuf_v.at[g % TBUF],
                out_hbm.at[g // SUBT, :, wid * SUBT + lax.rem(g, SUBT)],
                ssem.at[g % TBUF],
            ).wait()

    return _gather_scale


def kernel(x, input_embedding):
    xt = x.T
    lin = _build()(xt, input_embedding)
    out = lin.transpose(0, 1, 3, 2, 4).reshape(HIST, EMBED_DIM, BATCH)
    return out.transpose(2, 0, 1)


# speedup vs baseline: 17.8351x; 17.8351x over previous
import functools

import jax
import jax.numpy as jnp
from jax import lax
from jax.experimental import pallas as pl
from jax.experimental.pallas import tpu as pltpu
from jax.experimental.pallas import tpu_sc as plsc

BATCH = 16384
HIST = 50
EMBED_DIM = 64
NUM_CORES = 2
NUM_SUBCORES = 16
NUM_WORKERS = NUM_CORES * NUM_SUBCORES
BWIN = BATCH // NUM_WORKERS
CHUNK = 128
SUBT = BWIN // CHUNK
CPW = HIST * SUBT
BTILES = BATCH // CHUNK
SCALE = 8.0
LANES = 16
NBUF = 8
AHEAD = 6
TBUF = 4


@functools.cache
def _build():
    mesh = plsc.VectorSubcoreMesh(core_axis_name="c", subcore_axis_name="s")

    @functools.partial(
        pl.kernel,
        mesh=mesh,
        out_type=jax.ShapeDtypeStruct(
            (HIST, EMBED_DIM // 8, BTILES, 8, CHUNK), jnp.float32
        ),
        scratch_types=[
            pltpu.VMEM((CPW, CHUNK), jnp.int32),
            pltpu.VMEM((NBUF, CHUNK, EMBED_DIM), jnp.float32),
            pltpu.VMEM((TBUF, 8, 8, CHUNK), jnp.float32),
            pltpu.SemaphoreType.DMA,
            pltpu.SemaphoreType.DMA((NBUF,)),
            pltpu.SemaphoreType.DMA((TBUF,)),
        ],
        compiler_params=pltpu.CompilerParams(
            use_tc_tiling_on_sc=False, needs_layout_passes=False
        ),
    )
    def _gather_scale(xt_hbm, tab_hbm, out_hbm, idx_v, rows_v, tbuf_v,
                      isem, gsem, ssem):
        wid = lax.axis_index("s") * NUM_CORES + lax.axis_index("c")
        b_lo = wid * BWIN

        def stage(g, _):
            h = g // SUBT
            b0 = b_lo + lax.rem(g, SUBT) * CHUNK
            pltpu.async_copy(xt_hbm.at[h, pl.ds(b0, CHUNK)], idx_v.at[g], isem)
            return _

        lax.fori_loop(0, CPW, stage, 0)

        def stage_wait(g, _):
            pltpu.make_async_copy(
                xt_hbm.at[0, pl.ds(0, CHUNK)], idx_v.at[0], isem
            ).wait()
            return _

        lax.fori_loop(0, CPW, stage_wait, 0)

        def gather(g, b):
            pltpu.async_copy(tab_hbm.at[idx_v.at[g]], rows_v.at[b], gsem.at[b])

        for b in range(AHEAD):
            gather(b, b)

        lane_ids = [lax.iota(jnp.int32, LANES) + kb * LANES
                    for kb in range(CHUNK // LANES)]

        def outer(go):
            for b in range(NBUF):
                g = go + b
                h = g // SUBT
                btile = wid * SUBT + lax.rem(g, SUBT)
                tb = b % TBUF

                pltpu.make_async_copy(
                    tab_hbm.at[idx_v.at[g]], rows_v.at[b], gsem.at[b]
                ).wait()

                @pl.when(g >= TBUF)
                def _():
                    gp = g - TBUF
                    pltpu.make_async_copy(
                        tbuf_v.at[tb],
                        out_hbm.at[gp // SUBT, :,
                                   wid * SUBT + lax.rem(gp, SUBT)],
                        ssem.at[tb],
                    ).wait()

                for db in range(EMBED_DIM // LANES):

                    def c_body(c, rot, _b=b, _tb=tb, _db=db):
                        col = rot + _db * LANES
                        dt = col >> 3
                        dl = col & 7
                        for kb in range(CHUNK // LANES):
                            v = plsc.load_gather(
                                rows_v.at[_b], [lane_ids[kb], col]
                            )
                            plsc.store_scatter(
                                tbuf_v.at[_tb],
                                [dt, dl, lane_ids[kb]],
                                v * SCALE,
                            )
                        return (rot + 1) & (LANES - 1)

                    lax.fori_loop(0, LANES, c_body, lane_ids[0])

                pltpu.async_copy(
                    tbuf_v.at[tb], out_hbm.at[h, :, btile], ssem.at[tb]
                )

                gn = g + AHEAD

                @pl.when(gn < CPW)
                def _():
                    gather(gn, (b + AHEAD) % NBUF)

        pl.loop(0, CPW, step=NBUF)(outer)

        for t in range(TBUF):
            g = CPW - TBUF + t
            pltpu.make_async_copy(
                tbuf_v.at[g % TBUF],
                out_hbm.at[g // SUBT, :, wid * SUBT + lax.rem(g, SUBT)],
                ssem.at[g % TBUF],
            ).wait()

    return _gather_scale


def kernel(x, input_embedding):
    xt = x.T
    lin = _build()(xt, input_embedding)
    out = lin.transpose(0, 1, 3, 2, 4).reshape(HIST, EMBED_DIM, BATCH)
    return out.transpose(2, 0, 1)
